# Initial kernel scaffold; baseline (speedup 1.0000x reference)
import jax
import jax.numpy as jnp
from jax.experimental import pallas as pl


def kernel(a, b, Ea, Eb, W, bias):
    raise NotImplementedError("write your pallas kernel here")


# speedup vs baseline: 212.7682x; 212.7682x over previous
import functools

import jax
import jax.numpy as jnp
from jax import lax
from jax.experimental import pallas as pl
from jax.experimental.pallas import tpu as pltpu
from jax.experimental.pallas import tpu_sc as plsc

B = 16384
L = 200
NC = 2
NS = 16
LANES = 16
NW = NC * NS
ROWS_PER_W = B // NW
CHUNK = 128
NCHUNK = ROWS_PER_W // CHUNK
GROUPS = CHUNK // LANES
UNROLL = 8


def _sc_body(a_hbm, b_hbm, p_hbm, out_hbm, a_buf, b_buf, p_buf,
             tab_a, tab_b, out_buf):
    wid = lax.axis_index("s") * NC + lax.axis_index("c")
    base = wid * ROWS_PER_W
    ebase = base * L

    pltpu.sync_copy(p_hbm, p_buf)

    def prow(r):
        return p_buf[pl.ds(r * LANES, LANES)]

    tab_a[...] = prow(0) * prow(6) + prow(1) * prow(7) + prow(2) * prow(8)
    tab_b[...] = prow(3) * prow(9) + prow(4) * prow(10) + prow(5) * prow(11)
    acc0 = prow(12)

    iota200 = lax.iota(jnp.int32, LANES) * L

    for c in range(NCHUNK):
        pltpu.sync_copy(a_hbm.at[pl.ds(ebase + c * CHUNK * L, CHUNK * L)],
                        a_buf)
        pltpu.sync_copy(b_hbm.at[pl.ds(ebase + c * CHUNK * L, CHUNK * L)],
                        b_buf)
        for g in range(GROUPS):
            rows = iota200 + (g * LANES * L)

            def body(i, acc, rows=rows):
                flat0 = rows + i * UNROLL
                for u in range(UNROLL):
                    flat = flat0 + u
                    av = plsc.load_gather(a_buf, [flat])
                    bv = plsc.load_gather(b_buf, [flat])
                    acc = (acc + plsc.load_gather(tab_a, [av])
                           + plsc.load_gather(tab_b, [bv]))
                return acc

            acc = lax.fori_loop(0, L // UNROLL, body, acc0)
            out_buf[pl.ds(c * CHUNK + g * LANES, LANES)] = acc

    pltpu.sync_copy(out_buf, out_hbm.at[pl.ds(base, ROWS_PER_W)])


_sc_call = pl.kernel(
    _sc_body,
    out_type=jax.ShapeDtypeStruct((B,), jnp.float32),
    mesh=plsc.VectorSubcoreMesh(core_axis_name="c", subcore_axis_name="s"),
    compiler_params=pltpu.CompilerParams(needs_layout_passes=False),
    scratch_types=[
        pltpu.VMEM((CHUNK * L,), jnp.int32),
        pltpu.VMEM((CHUNK * L,), jnp.int32),
        pltpu.VMEM((13 * LANES,), jnp.float32),
        pltpu.VMEM((LANES,), jnp.float32),
        pltpu.VMEM((LANES,), jnp.float32),
        pltpu.VMEM((ROWS_PER_W,), jnp.float32),
    ],
)


@jax.jit
def kernel(a, b, Ea, Eb, W, bias):
    P = jnp.zeros((13, LANES), jnp.float32)
    P = P.at[0:3, 0:10].set(Ea.T)
    P = P.at[3:6, 0:10].set(Eb.T)
    P = P.at[6:12, :].set(jnp.broadcast_to(W.reshape(6, 1), (6, LANES)))
    P = P.at[12, :].set(bias[0])
    out = _sc_call(a.reshape(-1), b.reshape(-1), P.reshape(-1))
    return out.reshape(B, 1)


# speedup vs baseline: 234.2345x; 1.1009x over previous
import functools

import jax
import jax.numpy as jnp
from jax import lax
from jax.experimental import pallas as pl
from jax.experimental.pallas import tpu as pltpu
from jax.experimental.pallas import tpu_sc as plsc

B = 16384
L = 200
NC = 2
NS = 16
LANES = 16
NW = NC * NS
ROWS_PER_W = B // NW
CHUNK = 128
NCHUNK = ROWS_PER_W // CHUNK
GROUPS = CHUNK // LANES
UNROLL = 8


def _sc_body_v2(a_hbm, b_hbm, p_hbm, out_hbm, a_buf0, a_buf1, b_buf0, b_buf1,
                p_buf, tab_a, tab_b, out_buf, sem_a0, sem_a1, sem_b0, sem_b1):
    wid = lax.axis_index("s") * NC + lax.axis_index("c")
    base = wid * ROWS_PER_W
    ebase = base * L

    pltpu.sync_copy(p_hbm, p_buf)

    def prow(r):
        return p_buf[pl.ds(r * LANES, LANES)]

    tab_a[...] = prow(0) * prow(6) + prow(1) * prow(7) + prow(2) * prow(8)
    tab_b[...] = prow(3) * prow(9) + prow(4) * prow(10) + prow(5) * prow(11)
    acc_init = prow(12)
    zero = jnp.zeros((LANES,), jnp.float32)

    iota200 = lax.iota(jnp.int32, LANES) * L
    a_bufs = (a_buf0, a_buf1)
    b_bufs = (b_buf0, b_buf1)
    sems_a = (sem_a0, sem_a1)
    sems_b = (sem_b0, sem_b1)

    def start_chunk(c):
        slot = c % 2
        src = pl.ds(ebase + c * CHUNK * L, CHUNK * L)
        return (pltpu.async_copy(a_hbm.at[src], a_bufs[slot], sems_a[slot]),
                pltpu.async_copy(b_hbm.at[src], b_bufs[slot], sems_b[slot]))

    pending = start_chunk(0)
    for c in range(NCHUNK):
        nxt = start_chunk(c + 1) if c + 1 < NCHUNK else ()
        for cp in pending:
            cp.wait()
        pending = nxt
        a_buf = a_bufs[c % 2]
        b_buf = b_bufs[c % 2]
        for g in range(GROUPS):
            rows = iota200 + (g * LANES * L)

            @plsc.parallel_loop(0, L, step=UNROLL, unroll=2,
                                carry=(acc_init, zero, zero, zero))
            def body(l0, accs, rows=rows, a_buf=a_buf, b_buf=b_buf):
                a0, a1, a2, a3 = accs
                flat0 = rows + l0
                t = []
                for u in range(UNROLL):
                    av = plsc.load_gather(a_buf, [flat0 + u])
                    bv = plsc.load_gather(b_buf, [flat0 + u])
                    t.append(plsc.load_gather(tab_a, [av]))
                    t.append(plsc.load_gather(tab_b, [bv]))
                a0 = a0 + (t[0] + t[4]) + (t[8] + t[12])
                a1 = a1 + (t[1] + t[5]) + (t[9] + t[13])
                a2 = a2 + (t[2] + t[6]) + (t[10] + t[14])
                a3 = a3 + (t[3] + t[7]) + (t[11] + t[15])
                return a0, a1, a2, a3

            r0, r1, r2, r3 = body
            out_buf[pl.ds(c * CHUNK + g * LANES, LANES)] = (
                (r0 + r1) + (r2 + r3))

    pltpu.sync_copy(out_buf, out_hbm.at[pl.ds(base, ROWS_PER_W)])


_sc_call = pl.kernel(
    _sc_body_v2,
    out_type=jax.ShapeDtypeStruct((B,), jnp.float32),
    mesh=plsc.VectorSubcoreMesh(core_axis_name="c", subcore_axis_name="s"),
    compiler_params=pltpu.CompilerParams(needs_layout_passes=False),
    scratch_types=[
        pltpu.VMEM((CHUNK * L,), jnp.int32),
        pltpu.VMEM((CHUNK * L,), jnp.int32),
        pltpu.VMEM((CHUNK * L,), jnp.int32),
        pltpu.VMEM((CHUNK * L,), jnp.int32),
        pltpu.VMEM((13 * LANES,), jnp.float32),
        pltpu.VMEM((LANES,), jnp.float32),
        pltpu.VMEM((LANES,), jnp.float32),
        pltpu.VMEM((ROWS_PER_W,), jnp.float32),
        pltpu.SemaphoreType.DMA,
        pltpu.SemaphoreType.DMA,
        pltpu.SemaphoreType.DMA,
        pltpu.SemaphoreType.DMA,
    ],
)


@jax.jit
def kernel(a, b, Ea, Eb, W, bias):
    P = jnp.zeros((13, LANES), jnp.float32)
    P = P.at[0:3, 0:10].set(Ea.T)
    P = P.at[3:6, 0:10].set(Eb.T)
    P = P.at[6:12, :].set(jnp.broadcast_to(W.reshape(6, 1), (6, LANES)))
    P = P.at[12, :].set(bias[0])
    out = _sc_call(a.reshape(-1), b.reshape(-1), P.reshape(-1))
    return out.reshape(B, 1)
